# baseline (speedup 1.0000x reference)
import jax
import jax.numpy as jnp
from jax.experimental import pallas as pl
from jax.experimental.pallas import tpu as pltpu


_NS = 8


def _k(gu_hbm, gi_hbm, xui_ref, ub, vb, su, sv):
    B = gu_hbm.shape[0]
    CH = B // _NS
    def cps(k):
        sl = pl.ds(k * CH, CH)
        return (
            pltpu.make_async_copy(gu_hbm.at[sl, :], ub.at[sl, :], su.at[k]),
            pltpu.make_async_copy(gi_hbm.at[sl, :], vb.at[sl, :], sv.at[k]),
        )
    for k in range(_NS):
        for cp in cps(k):
            cp.start()
    for k in range(_NS):
        for cp in cps(k):
            cp.wait()
    xui_ref[:] = jnp.sum(ub[:] * vb[:], axis=1)


def kernel(gu, gi):
    B, D = gu.shape
    xui = pl.pallas_call(
        _k,
        in_specs=[
            pl.BlockSpec(memory_space=pl.ANY),
            pl.BlockSpec(memory_space=pl.ANY),
        ],
        out_specs=pl.BlockSpec(memory_space=pltpu.MemorySpace.VMEM),
        out_shape=jax.ShapeDtypeStruct((B,), jnp.float32),
        scratch_shapes=[
            pltpu.MemorySpace.VMEM((B, D), jnp.float32),
            pltpu.MemorySpace.VMEM((B, D), jnp.float32),
            pltpu.SemaphoreType.DMA((_NS,)),
            pltpu.SemaphoreType.DMA((_NS,)),
        ],
    )(gu, gi)
    return (xui, gu, gi)
